# baseline (speedup 1.0000x reference)
import functools

import jax
import jax.numpy as jnp
from jax.experimental import pallas as pl
from jax.experimental.pallas import tpu as pltpu

_B = 4
_K = 3


def _matmul3_kernel(y_ref, s1_ref, s2_ref, w_ref, o_ref):
    y = y_ref[...]
    xt = jnp.concatenate([y, -s1_ref[...], 2.0 * s2_ref[...] - y], axis=1)
    o_ref[...] = jnp.dot(xt, w_ref[...], preferred_element_type=jnp.float32)


def _cheb_combine(y, s1, s2, W):
    B, V, C = y.shape
    Cout = W.shape[1]
    yr = y.reshape(B * V, C)
    s1r = s1.reshape(B * V, C)
    s2r = s2.reshape(B * V, C)
    N = B * V
    RB = min(N, 1024)
    grid = (N // RB,)
    out = pl.pallas_call(
        _matmul3_kernel,
        grid=grid,
        in_specs=[
            pl.BlockSpec((RB, C), lambda i: (i, 0)),
            pl.BlockSpec((RB, C), lambda i: (i, 0)),
            pl.BlockSpec((RB, C), lambda i: (i, 0)),
            pl.BlockSpec((3 * C, Cout), lambda i: (0, 0)),
        ],
        out_specs=pl.BlockSpec((RB, Cout), lambda i: (i, 0)),
        out_shape=jax.ShapeDtypeStruct((N, Cout), jnp.float32),
    )(yr, s1r, s2r, W)
    return out.reshape(B, V, Cout)


def _matvec(y, src, dst, nw):
    g = y[:, src, :] * nw[None, :, None]
    return jnp.zeros_like(y).at[:, dst, :].add(g)


def _edge_norm(src, dst, w, V):
    deg = jnp.zeros((V,), w.dtype).at[dst].add(w) + 1e-6
    dis = jax.lax.rsqrt(deg)
    return w * dis[src] * dis[dst]


def _cheb_conv(x, src, dst, w, W):
    V = x.shape[1]
    nw = _edge_norm(src, dst, w, V)
    s1 = _matvec(x, src, dst, nw)
    s2 = _matvec(s1, src, dst, nw)
    return _cheb_combine(x, s1, s2, W)


def _bn_relu(x, g, b):
    m = jnp.mean(x, axis=(0, 1))
    v = jnp.var(x, axis=(0, 1))
    return jax.nn.relu((x - m) / jnp.sqrt(v + 1e-5) * g + b)


def _pool(x):
    Bn, V, C = x.shape
    return x.reshape(Bn, V // 4, 4, C).mean(axis=2)


def kernel(x, src5, dst5, wt5, src4, dst4, wt4, src3, dst3, wt3, src2, dst2, wt2, src1, dst1, wt1, src0, dst0, wt0, W5a, g5a, b5a, W5b, g5b, b5b, W4, g4, b4, W3, g3, b3, W2, g2, b2, W1, g1, b1, W0):
    h = _bn_relu(_cheb_conv(x, src5, dst5, wt5, W5a), g5a, b5a)
    h5 = _bn_relu(_cheb_conv(h, src5, dst5, wt5, W5b), g5b, b5b)
    h4 = _bn_relu(_cheb_conv(_pool(h5), src4, dst4, wt4, W4), g4, b4)
    h3 = _bn_relu(_cheb_conv(_pool(h4), src3, dst3, wt3, W3), g3, b3)
    h2 = _bn_relu(_cheb_conv(_pool(h3), src2, dst2, wt2, W2), g2, b2)
    h1 = _bn_relu(_cheb_conv(_pool(h2), src1, dst1, wt1, W1), g1, b1)
    h0 = _cheb_conv(_pool(h1), src0, dst0, wt0, W0)
    return (h0, h1, h2, h3, h4)


# speedup vs baseline: 24.0942x; 24.0942x over previous
import functools

import jax
import jax.numpy as jnp
from jax import lax
from jax.experimental import pallas as pl
from jax.experimental.pallas import tpu as pltpu
from jax.experimental.pallas import tpu_sc as plsc

_B = 4


def _zbuf_rows(v16, c, budget_bytes=131072):
    for k in range(1, v16 + 1):
        if v16 % k == 0 and (v16 // k) * c * 4 <= budget_bytes:
            return v16 // k
    return 1


def _best_win(ew, mult):
    for d in range(min(128, ew), 0, -1):
        if ew % d == 0 and d % mult == 0:
            return d
    raise ValueError((ew, mult))


def _n_tiles(E):
    for nt in range(16, 0, -1):
        if E % nt == 0 and (E // nt) % 16 == 0:
            return nt
    raise ValueError(E)


_MESH = plsc.VectorSubcoreMesh(core_axis_name="c", subcore_axis_name="s")


@functools.partial(jax.jit, static_argnames=("V", "E"))
def _sc_degree(dst, w, *, V, E):
    half = E // 2
    nt = _n_tiles(half)
    ew = half // nt
    win = _best_win(ew, 8)
    n_win = ew // win
    nf = None
    for n in (16, 12, 8, 6, 4, 3, 2, 1):
        if V % n == 0 and (V // n) % 16 == 0:
            nf = n
            break
    fc = V // nf

    @functools.partial(
        pl.kernel,
        mesh=_MESH,
        compiler_params=pltpu.CompilerParams(
            needs_layout_passes=False, use_tc_tiling_on_sc=False),
        out_type=jax.ShapeDtypeStruct((2 * V,), jnp.float32),
        scratch_types=[
            pltpu.VMEM_SHARED((V,), jnp.float32),
            pltpu.VMEM((win,), jnp.int32),
            pltpu.VMEM((win,), jnp.float32),
            pltpu.VMEM((fc,), jnp.float32),
        ],
    )
    def k(dst_h, w_h, out_h, acc, didx_v, wv, zbuf):
        c = lax.axis_index("c")
        s = lax.axis_index("s")
        for j in range(fc // 16):
            zbuf[pl.ds(16 * j, 16)] = jnp.zeros((16,), jnp.float32)

        @pl.when(s < nf)
        def _zero():
            pltpu.sync_copy(zbuf, acc.at[pl.ds(s * fc, fc)])

        plsc.subcore_barrier()

        @pl.when(s < nt)
        def _work():
            base = c * half + s * ew

            @pl.loop(0, n_win)
            def _win(wi):
                eo = base + wi * win
                pltpu.sync_copy(dst_h.at[pl.ds(eo, win)], didx_v)
                pltpu.sync_copy(w_h.at[pl.ds(eo, win)], wv)
                pltpu.sync_copy(wv, acc.at[didx_v], add=True)

        plsc.subcore_barrier()

        @pl.when(s < nf)
        def _flush():
            pltpu.sync_copy(
                acc.at[pl.ds(s * fc, fc)],
                out_h.at[pl.ds(c * V + s * fc, fc)],
            )

    return k(dst, w).reshape(2, V)


def _dis_kernel(deg_ref, o_ref):
    d = deg_ref[0:1, :] + deg_ref[1:2, :] + 1e-6
    o_ref[...] = 1.0 / jnp.sqrt(d)


def _tc_dis(deg_partial, V):
    return pl.pallas_call(
        _dis_kernel,
        out_shape=jax.ShapeDtypeStruct((1, V), jnp.float32),
    )(deg_partial).reshape(V)


@functools.partial(jax.jit, static_argnames=("V", "E"))
def _sc_edge_norm(src, dst, w, dis, *, V, E):
    n_workers = None
    for n in (32, 16, 8, 4, 2, 1):
        if E % n == 0 and (E // n) % 16 == 0:
            n_workers = n
            break
    ew = E // n_workers

    @functools.partial(
        pl.kernel,
        mesh=_MESH,
        compiler_params=pltpu.CompilerParams(needs_layout_passes=False),
        out_type=jax.ShapeDtypeStruct((E,), jnp.float32),
        scratch_types=[
            pltpu.VMEM((V,), jnp.float32),
            pltpu.VMEM((ew,), jnp.int32),
            pltpu.VMEM((ew,), jnp.int32),
            pltpu.VMEM((ew,), jnp.float32),
            pltpu.VMEM((ew,), jnp.float32),
        ],
    )
    def k(src_h, dst_h, w_h, dis_h, out_h, dis_v, sv, dv, wv, ov):
        c = lax.axis_index("c")
        s = lax.axis_index("s")
        if n_workers == 32:
            wid = s * 2 + c
            active = s >= 0
        else:
            wid = s
            active = (c == 0) & (s < n_workers)
        pltpu.sync_copy(dis_h, dis_v)

        @pl.when(active)
        def _work():
            base = wid * ew
            pltpu.sync_copy(src_h.at[pl.ds(base, ew)], sv)
            pltpu.sync_copy(dst_h.at[pl.ds(base, ew)], dv)
            pltpu.sync_copy(w_h.at[pl.ds(base, ew)], wv)

            @pl.loop(0, ew // 16)
            def _chunk(j):
                sl = pl.ds(16 * j, 16)
                a = plsc.load_gather(dis_v, [sv[sl]])
                b = plsc.load_gather(dis_v, [dv[sl]])
                ov[sl] = wv[sl] * a * b

            pltpu.sync_copy(ov, out_h.at[pl.ds(base, ew)])

    return k(src, dst, w, dis)


@functools.partial(jax.jit, static_argnames=("V", "C", "E", "n_ch"))
def _sc_matvec(table3, src, dst, nw, *, V, C, E, n_ch=1):
    Cg = C // n_ch
    nt = _n_tiles(E)
    ew = E // nt
    win = _best_win(ew, 16)
    n_win = ew // win
    v16 = V // 16
    zr = _zbuf_rows(v16, Cg)

    @functools.partial(
        pl.kernel,
        mesh=_MESH,
        compiler_params=pltpu.CompilerParams(
            needs_layout_passes=False, use_tc_tiling_on_sc=False),
        out_type=jax.ShapeDtypeStruct((n_ch, _B * V, Cg), jnp.float32),
        scratch_types=[
            pltpu.VMEM_SHARED((V, Cg), jnp.float32),
            pltpu.VMEM((win,), jnp.int32),
            pltpu.VMEM((win,), jnp.int32),
            pltpu.VMEM((win,), jnp.float32),
            pltpu.VMEM((win, Cg), jnp.float32),
            pltpu.VMEM((zr, Cg), jnp.float32),
            pltpu.SemaphoreType.DMA,
        ],
    )
    def k(tab_h, src_h, dst_h, nw_h, out_h, acc, idx_v, didx_v, nw_v, rows_v,
          zbuf, sem):
        c = lax.axis_index("c")
        s = lax.axis_index("s")
        for r in range(zr):
            for cc in range(Cg // 16):
                zbuf[r, pl.ds(16 * cc, 16)] = jnp.zeros((16,), jnp.float32)

        for ch in range(n_ch):
            for p in range(2):
                b = 2 * c + p

                @pl.loop(0, v16 // zr)
                def _zero(kk):
                    pltpu.sync_copy(zbuf, acc.at[pl.ds(s * v16 + kk * zr, zr)])

                plsc.subcore_barrier()

                @pl.when(s < nt)
                def _work():
                    @pl.loop(0, n_win)
                    def _win(wi):
                        eo = s * ew + wi * win
                        pltpu.sync_copy(src_h.at[pl.ds(eo, win)], idx_v)
                        pltpu.sync_copy(dst_h.at[pl.ds(eo, win)], didx_v)
                        pltpu.sync_copy(nw_h.at[pl.ds(eo, win)], nw_v)
                        boff = jnp.full((16,), b * V, jnp.int32)
                        for j in range(win // 16):
                            sl = pl.ds(16 * j, 16)
                            idx_v[sl] = idx_v[sl] + boff
                        pltpu.async_copy(
                            tab_h.at[ch].at[idx_v], rows_v, sem).wait()

                        @pl.loop(0, win // 16)
                        def _scale(j):
                            for i in range(16):
                                e = 16 * j + i
                                spl = plsc.load_gather(
                                    nw_v, [jnp.full((16,), e, jnp.int32)]
                                )
                                for cc in range(Cg // 16):
                                    sl = pl.ds(16 * cc, 16)
                                    rows_v[e, sl] = rows_v[e, sl] * spl

                        pltpu.sync_copy(rows_v, acc.at[didx_v], add=True)

                plsc.subcore_barrier()
                pltpu.sync_copy(
                    acc.at[pl.ds(s * v16, v16)],
                    out_h.at[ch, pl.ds(b * V + s * v16, v16)],
                )
                plsc.subcore_barrier()

    return k(table3, src, dst, nw)


def _merge(ref):
    v = ref[...]
    n_ch, R, Cg = v.shape
    if n_ch == 1:
        return v.reshape(R, Cg)
    return jnp.transpose(v, (1, 0, 2)).reshape(R, n_ch * Cg)


def _mm_stats_kernel(y_ref, s1_ref, s2_ref, w_ref, o_ref, st_ref, st_acc):
    i = pl.program_id(0)
    y = _merge(y_ref)
    xt = jnp.concatenate([y, -_merge(s1_ref), 2.0 * _merge(s2_ref) - y],
                         axis=1)
    h = jnp.dot(xt, w_ref[...], preferred_element_type=jnp.float32)
    o_ref[...] = h
    part = jnp.stack([jnp.sum(h, axis=0), jnp.sum(h * h, axis=0)])

    @pl.when(i == 0)
    def _():
        st_acc[...] = jnp.zeros_like(st_acc)

    st_acc[...] += part

    @pl.when(i == pl.num_programs(0) - 1)
    def _():
        st_ref[...] = st_acc[...]


def _mm_kernel(y_ref, s1_ref, s2_ref, w_ref, o_ref):
    y = _merge(y_ref)
    xt = jnp.concatenate([y, -_merge(s1_ref), 2.0 * _merge(s2_ref) - y],
                         axis=1)
    o_ref[...] = jnp.dot(xt, w_ref[...], preferred_element_type=jnp.float32)


def _tc_combine(y, s1, s2, W, with_stats):
    n_ch, N, Cg = y.shape
    C = n_ch * Cg
    Cout = W.shape[1]
    RB = min(N, 1024)
    grid = (N // RB,)
    spec3 = pl.BlockSpec((n_ch, RB, Cg), lambda i: (0, i, 0))
    in_specs = [spec3, spec3, spec3,
                pl.BlockSpec((3 * C, Cout), lambda i: (0, 0))]
    if with_stats:
        return pl.pallas_call(
            _mm_stats_kernel,
            grid=grid,
            in_specs=in_specs,
            out_specs=[
                pl.BlockSpec((RB, Cout), lambda i: (i, 0)),
                pl.BlockSpec((2, Cout), lambda i: (0, 0)),
            ],
            out_shape=[
                jax.ShapeDtypeStruct((N, Cout), jnp.float32),
                jax.ShapeDtypeStruct((2, Cout), jnp.float32),
            ],
            scratch_shapes=[pltpu.VMEM((2, Cout), jnp.float32)],
        )(y, s1, s2, W)
    return pl.pallas_call(
        _mm_kernel,
        grid=grid,
        in_specs=in_specs,
        out_specs=pl.BlockSpec((RB, Cout), lambda i: (i, 0)),
        out_shape=jax.ShapeDtypeStruct((N, Cout), jnp.float32),
    )(y, s1, s2, W)


def _bn_body(h_ref, st_ref, g_ref, b_ref, ninv):
    st = st_ref[...]
    m = st[0:1, :] * ninv
    var = st[1:2, :] * ninv - m * m
    scale = g_ref[...] / jnp.sqrt(var + 1e-5)
    return jax.nn.relu((h_ref[...] - m) * scale + b_ref[...])


def _bn_kernel_full_pool(h_ref, st_ref, g_ref, b_ref, on_ref, op_ref, *, ninv):
    hn = _bn_body(h_ref, st_ref, g_ref, b_ref, ninv)
    on_ref[...] = hn
    R = hn.shape[0]
    op_ref[...] = jnp.mean(hn.reshape(R // 4, 4, -1), axis=1)


def _bn_kernel_pool(h_ref, st_ref, g_ref, b_ref, op_ref, *, ninv):
    hn = _bn_body(h_ref, st_ref, g_ref, b_ref, ninv)
    R = hn.shape[0]
    op_ref[...] = jnp.mean(hn.reshape(R // 4, 4, -1), axis=1)


def _bn_kernel_split(h_ref, st_ref, g_ref, b_ref, on_ref, *, ninv, out_nch):
    hn = _bn_body(h_ref, st_ref, g_ref, b_ref, ninv)
    R, C = hn.shape
    on_ref[...] = jnp.transpose(
        hn.reshape(R, out_nch, C // out_nch), (1, 0, 2))


def _bn_kernel_full(h_ref, st_ref, g_ref, b_ref, on_ref, *, ninv):
    on_ref[...] = _bn_body(h_ref, st_ref, g_ref, b_ref, ninv)


def _tc_bn(h, stats, g, b, V, emit_full, emit_pool, full_nch=None):
    N, Cout = h.shape
    RB = min(V, 1024)
    grid = (N // RB,)
    gb = g.reshape(1, Cout)
    bb = b.reshape(1, Cout)
    ninv = 1.0 / N
    in_specs = [
        pl.BlockSpec((RB, Cout), lambda i: (i, 0)),
        pl.BlockSpec((2, Cout), lambda i: (0, 0)),
        pl.BlockSpec((1, Cout), lambda i: (0, 0)),
        pl.BlockSpec((1, Cout), lambda i: (0, 0)),
    ]
    full_spec = pl.BlockSpec((RB, Cout), lambda i: (i, 0))
    pool_spec = pl.BlockSpec((RB // 4, Cout), lambda i: (i, 0))
    full_shape = jax.ShapeDtypeStruct((N, Cout), jnp.float32)
    pool_shape = jax.ShapeDtypeStruct((N // 4, Cout), jnp.float32)
    if emit_full and emit_pool:
        return pl.pallas_call(
            functools.partial(_bn_kernel_full_pool, ninv=ninv),
            grid=grid, in_specs=in_specs,
            out_specs=[full_spec, pool_spec],
            out_shape=[full_shape, pool_shape],
        )(h, stats, gb, bb)
    if emit_pool:
        return None, pl.pallas_call(
            functools.partial(_bn_kernel_pool, ninv=ninv),
            grid=grid, in_specs=in_specs,
            out_specs=pool_spec, out_shape=pool_shape,
        )(h, stats, gb, bb)
    if full_nch is not None:
        cg = Cout // full_nch
        return pl.pallas_call(
            functools.partial(_bn_kernel_split, ninv=ninv, out_nch=full_nch),
            grid=grid, in_specs=in_specs,
            out_specs=pl.BlockSpec((full_nch, RB, cg), lambda i: (0, i, 0)),
            out_shape=jax.ShapeDtypeStruct((full_nch, N, cg), jnp.float32),
        )(h, stats, gb, bb), None
    return pl.pallas_call(
        functools.partial(_bn_kernel_full, ninv=ninv),
        grid=grid, in_specs=in_specs,
        out_specs=full_spec, out_shape=full_shape,
    )(h, stats, gb, bb), None


def _level_prep(src, dst, w, V):
    E = src.shape[0]
    deg_p = _sc_degree(dst, w, V=V, E=E)
    dis = _tc_dis(deg_p, V)
    return _sc_edge_norm(src, dst, w, dis, V=V, E=E)


def _conv(table3, src, dst, nw, V, C, n_ch=1):
    E = src.shape[0]
    s1 = _sc_matvec(table3, src, dst, nw, V=V, C=C, E=E, n_ch=n_ch)
    s2 = _sc_matvec(s1, src, dst, nw, V=V, C=C, E=E, n_ch=n_ch)
    return s1, s2


def kernel(x, src5, dst5, wt5, src4, dst4, wt4, src3, dst3, wt3, src2, dst2, wt2, src1, dst1, wt1, src0, dst0, wt0, W5a, g5a, b5a, W5b, g5b, b5b, W4, g4, b4, W3, g3, b3, W2, g2, b2, W1, g1, b1, W0):
    B, V5, C0 = x.shape
    vs = [V5 // (4 ** i) for i in range(6)]

    nw5 = _level_prep(src5, dst5, wt5, vs[0])

    t = x.reshape(1, B * vs[0], C0)
    s1, s2 = _conv(t, src5, dst5, nw5, vs[0], C0)
    h, st = _tc_combine(t, s1, s2, W5a, True)
    t5a, _ = _tc_bn(h, st, g5a, b5a, vs[0], True, False, full_nch=2)

    s1, s2 = _conv(t5a, src5, dst5, nw5, vs[0], 32, n_ch=2)
    h, st = _tc_combine(t5a, s1, s2, W5b, True)
    _, t4 = _tc_bn(h, st, g5b, b5b, vs[0], False, True)

    outs = []
    params = [
        (W4, g4, b4, src4, dst4, wt4),
        (W3, g3, b3, src3, dst3, wt3),
        (W2, g2, b2, src2, dst2, wt2),
        (W1, g1, b1, src1, dst1, wt1),
    ]
    t_cur = t4
    for li, (Wl, gl, bl, srcl, dstl, wl) in enumerate(params):
        Vl = vs[1 + li]
        Cin = Wl.shape[0] // 3
        t3 = t_cur.reshape(1, B * Vl, Cin)
        nwl = _level_prep(srcl, dstl, wl, Vl)
        s1, s2 = _conv(t3, srcl, dstl, nwl, Vl, Cin)
        h, st = _tc_combine(t3, s1, s2, Wl, True)
        hfull, t_next = _tc_bn(h, st, gl, bl, Vl, True, True)
        outs.append(hfull.reshape(B, Vl, Wl.shape[1]))
        t_cur = t_next

    V0 = vs[5]
    t3 = t_cur.reshape(1, B * V0, 512)
    nw0 = _level_prep(src0, dst0, wt0, V0)
    s1, s2 = _conv(t3, src0, dst0, nw0, V0, 512)
    h0 = _tc_combine(t3, s1, s2, W0, False)

    h4, h3, h2, h1 = outs
    return (h0.reshape(B, V0, 512), h1, h2, h3, h4)


# speedup vs baseline: 39.4665x; 1.6380x over previous
import functools

import jax
import jax.numpy as jnp
from jax import lax
from jax.experimental import pallas as pl
from jax.experimental.pallas import tpu as pltpu
from jax.experimental.pallas import tpu_sc as plsc

_B = 4


def _zbuf_rows(v16, c, budget_bytes=131072):
    for k in range(1, v16 + 1):
        if v16 % k == 0 and (v16 // k) * c * 4 <= budget_bytes:
            return v16 // k
    return 1


def _best_win(ew, mult):
    for d in range(min(128, ew), 0, -1):
        if ew % d == 0 and d % mult == 0:
            return d
    raise ValueError((ew, mult))


def _n_tiles(E):
    for nt in range(16, 0, -1):
        if E % nt == 0 and (E // nt) % 16 == 0:
            return nt
    raise ValueError(E)


_MESH = plsc.VectorSubcoreMesh(core_axis_name="c", subcore_axis_name="s")


@functools.partial(jax.jit, static_argnames=("V", "E"))
def _sc_degree(dst, w, *, V, E):
    half = E // 2
    nt = _n_tiles(half)
    ew = half // nt
    win = _best_win(ew, 8)
    n_win = ew // win
    nf = None
    for n in (16, 12, 8, 6, 4, 3, 2, 1):
        if V % n == 0 and (V // n) % 16 == 0:
            nf = n
            break
    fc = V // nf

    @functools.partial(
        pl.kernel,
        mesh=_MESH,
        compiler_params=pltpu.CompilerParams(
            needs_layout_passes=False, use_tc_tiling_on_sc=False),
        out_type=jax.ShapeDtypeStruct((2 * V,), jnp.float32),
        scratch_types=[
            pltpu.VMEM_SHARED((V,), jnp.float32),
            pltpu.VMEM((win,), jnp.int32),
            pltpu.VMEM((win,), jnp.float32),
            pltpu.VMEM((fc,), jnp.float32),
        ],
    )
    def k(dst_h, w_h, out_h, acc, didx_v, wv, zbuf):
        c = lax.axis_index("c")
        s = lax.axis_index("s")
        for j in range(fc // 16):
            zbuf[pl.ds(16 * j, 16)] = jnp.zeros((16,), jnp.float32)

        @pl.when(s < nf)
        def _zero():
            pltpu.sync_copy(zbuf, acc.at[pl.ds(s * fc, fc)])

        plsc.subcore_barrier()

        @pl.when(s < nt)
        def _work():
            base = c * half + s * ew

            @pl.loop(0, n_win)
            def _win(wi):
                eo = base + wi * win
                pltpu.sync_copy(dst_h.at[pl.ds(eo, win)], didx_v)
                pltpu.sync_copy(w_h.at[pl.ds(eo, win)], wv)
                pltpu.sync_copy(wv, acc.at[didx_v], add=True)

        plsc.subcore_barrier()

        @pl.when(s < nf)
        def _flush():
            pltpu.sync_copy(
                acc.at[pl.ds(s * fc, fc)],
                out_h.at[pl.ds(c * V + s * fc, fc)],
            )

    return k(dst, w).reshape(2, V)


def _dis_kernel(deg_ref, o_ref):
    d = deg_ref[0:1, :] + deg_ref[1:2, :] + 1e-6
    o_ref[...] = 1.0 / jnp.sqrt(d)


def _tc_dis(deg_partial, V):
    return pl.pallas_call(
        _dis_kernel,
        out_shape=jax.ShapeDtypeStruct((1, V), jnp.float32),
    )(deg_partial).reshape(V)


@functools.partial(jax.jit, static_argnames=("V", "E"))
def _sc_edge_norm(src, dst, w, dis, *, V, E):
    n_workers = None
    for n in (32, 16, 8, 4, 2, 1):
        if E % n == 0 and (E // n) % 16 == 0:
            n_workers = n
            break
    ew = E // n_workers

    @functools.partial(
        pl.kernel,
        mesh=_MESH,
        compiler_params=pltpu.CompilerParams(needs_layout_passes=False),
        out_type=jax.ShapeDtypeStruct((E,), jnp.float32),
        scratch_types=[
            pltpu.VMEM((V,), jnp.float32),
            pltpu.VMEM((ew,), jnp.int32),
            pltpu.VMEM((ew,), jnp.int32),
            pltpu.VMEM((ew,), jnp.float32),
            pltpu.VMEM((ew,), jnp.float32),
        ],
    )
    def k(src_h, dst_h, w_h, dis_h, out_h, dis_v, sv, dv, wv, ov):
        c = lax.axis_index("c")
        s = lax.axis_index("s")
        if n_workers == 32:
            wid = s * 2 + c
            active = s >= 0
        else:
            wid = s
            active = (c == 0) & (s < n_workers)
        pltpu.sync_copy(dis_h, dis_v)

        @pl.when(active)
        def _work():
            base = wid * ew
            pltpu.sync_copy(src_h.at[pl.ds(base, ew)], sv)
            pltpu.sync_copy(dst_h.at[pl.ds(base, ew)], dv)
            pltpu.sync_copy(w_h.at[pl.ds(base, ew)], wv)

            @pl.loop(0, ew // 16)
            def _chunk(j):
                sl = pl.ds(16 * j, 16)
                a = plsc.load_gather(dis_v, [sv[sl]])
                b = plsc.load_gather(dis_v, [dv[sl]])
                ov[sl] = wv[sl] * a * b

            pltpu.sync_copy(ov, out_h.at[pl.ds(base, ew)])

    return k(src, dst, w, dis)


@functools.partial(jax.jit, static_argnames=("V", "C", "E", "n_ch"))
def _sc_matvec(table3, src, dst, nw, *, V, C, E, n_ch=1):
    Cg = C // n_ch
    nt = _n_tiles(E)
    ew = E // nt
    win = _best_win(ew, 16)
    n_win = ew // win
    v16 = V // 16
    zr = _zbuf_rows(v16, Cg)
    pipelined = n_win >= 4 and n_win % 2 == 0

    @functools.partial(
        pl.kernel,
        mesh=_MESH,
        compiler_params=pltpu.CompilerParams(
            needs_layout_passes=False, use_tc_tiling_on_sc=False),
        out_type=jax.ShapeDtypeStruct((n_ch, _B * V, Cg), jnp.float32),
        scratch_types=[
            pltpu.VMEM_SHARED((V, Cg), jnp.float32),
            pltpu.VMEM((2, win), jnp.int32),
            pltpu.VMEM((2, win), jnp.int32),
            pltpu.VMEM((2, win), jnp.float32),
            pltpu.VMEM((2, win, Cg), jnp.float32),
            pltpu.VMEM((zr, Cg), jnp.float32),
            pltpu.SemaphoreType.DMA((2,)),
            pltpu.SemaphoreType.DMA((2,)),
        ],
    )
    def k(tab_h, src_h, dst_h, nw_h, out_h, acc, idx_v, didx_v, nw_v, rows_v,
          zbuf, sem_f, sem_g):
        c = lax.axis_index("c")
        s = lax.axis_index("s")
        for r in range(zr):
            for cc in range(Cg // 16):
                zbuf[r, pl.ds(16 * cc, 16)] = jnp.zeros((16,), jnp.float32)

        def fetch(q, wi):
            eo = s * ew + wi * win
            pltpu.async_copy(src_h.at[pl.ds(eo, win)], idx_v.at[q],
                             sem_f.at[q])
            pltpu.async_copy(dst_h.at[pl.ds(eo, win)], didx_v.at[q],
                             sem_f.at[q])
            pltpu.async_copy(nw_h.at[pl.ds(eo, win)], nw_v.at[q],
                             sem_f.at[q])

        def wait_fetch(q):
            pltpu.make_async_copy(src_h.at[pl.ds(0, win)], idx_v.at[q],
                                  sem_f.at[q]).wait()
            pltpu.make_async_copy(dst_h.at[pl.ds(0, win)], didx_v.at[q],
                                  sem_f.at[q]).wait()
            pltpu.make_async_copy(nw_h.at[pl.ds(0, win)], nw_v.at[q],
                                  sem_f.at[q]).wait()

        def gather(q, ch, b):
            boff = jnp.full((16,), b * V, jnp.int32)
            for j in range(win // 16):
                sl = pl.ds(16 * j, 16)
                idx_v[q, sl] = idx_v[q, sl] + boff
            pltpu.async_copy(tab_h.at[ch].at[idx_v.at[q]], rows_v.at[q],
                             sem_g.at[q])

        def wait_gather(q, ch):
            pltpu.make_async_copy(tab_h.at[ch].at[idx_v.at[q]],
                                  rows_v.at[q], sem_g.at[q]).wait()

        def scale(q):
            @pl.loop(0, win // 16)
            def _scale(j):
                for i in range(16):
                    e = 16 * j + i
                    spl = plsc.load_gather(
                        nw_v.at[q], [jnp.full((16,), e, jnp.int32)]
                    )
                    for cc in range(Cg // 16):
                        sl = pl.ds(16 * cc, 16)
                        rows_v[q, e, sl] = rows_v[q, e, sl] * spl

        def scatter(q):
            pltpu.sync_copy(rows_v.at[q], acc.at[didx_v.at[q]], add=True)

        for ch in range(n_ch):
            for p in range(2):
                b = 2 * c + p

                @pl.loop(0, v16 // zr)
                def _zero(kk):
                    pltpu.sync_copy(zbuf, acc.at[pl.ds(s * v16 + kk * zr, zr)])

                plsc.subcore_barrier()

                @pl.when(s < nt)
                def _work():
                    if pipelined:
                        fetch(0, 0)
                        wait_fetch(0)
                        gather(0, ch, b)
                        fetch(1, 1)

                        @pl.loop(0, n_win // 2)
                        def _pair(kk):
                            not_last = kk < n_win // 2 - 1
                            wait_gather(0, ch)
                            scale(0)
                            wait_fetch(1)
                            gather(1, ch, b)

                            @pl.when(not_last)
                            def _():
                                fetch(0, 2 * kk + 2)

                            scatter(0)
                            wait_gather(1, ch)
                            scale(1)

                            @pl.when(not_last)
                            def _():
                                wait_fetch(0)
                                gather(0, ch, b)
                                fetch(1, 2 * kk + 3)

                            scatter(1)
                    else:
                        @pl.loop(0, n_win)
                        def _win(wi):
                            fetch(0, wi)
                            wait_fetch(0)
                            gather(0, ch, b)
                            wait_gather(0, ch)
                            scale(0)
                            scatter(0)

                plsc.subcore_barrier()
                pltpu.sync_copy(
                    acc.at[pl.ds(s * v16, v16)],
                    out_h.at[ch, pl.ds(b * V + s * v16, v16)],
                )
                plsc.subcore_barrier()

    return k(table3, src, dst, nw)


def _merge(ref):
    v = ref[...]
    n_ch, R, Cg = v.shape
    if n_ch == 1:
        return v.reshape(R, Cg)
    return jnp.transpose(v, (1, 0, 2)).reshape(R, n_ch * Cg)


def _mm_stats_kernel(y_ref, s1_ref, s2_ref, w_ref, o_ref, st_ref, st_acc):
    i = pl.program_id(0)
    y = _merge(y_ref)
    xt = jnp.concatenate([y, -_merge(s1_ref), 2.0 * _merge(s2_ref) - y],
                         axis=1)
    h = jnp.dot(xt, w_ref[...], preferred_element_type=jnp.float32)
    o_ref[...] = h
    part = jnp.stack([jnp.sum(h, axis=0), jnp.sum(h * h, axis=0)])

    @pl.when(i == 0)
    def _():
        st_acc[...] = jnp.zeros_like(st_acc)

    st_acc[...] += part

    @pl.when(i == pl.num_programs(0) - 1)
    def _():
        st_ref[...] = st_acc[...]


def _mm_kernel(y_ref, s1_ref, s2_ref, w_ref, o_ref):
    y = _merge(y_ref)
    xt = jnp.concatenate([y, -_merge(s1_ref), 2.0 * _merge(s2_ref) - y],
                         axis=1)
    o_ref[...] = jnp.dot(xt, w_ref[...], preferred_element_type=jnp.float32)


def _tc_combine(y, s1, s2, W, with_stats):
    n_ch, N, Cg = y.shape
    C = n_ch * Cg
    Cout = W.shape[1]
    RB = min(N, 1024)
    grid = (N // RB,)
    spec3 = pl.BlockSpec((n_ch, RB, Cg), lambda i: (0, i, 0))
    in_specs = [spec3, spec3, spec3,
                pl.BlockSpec((3 * C, Cout), lambda i: (0, 0))]
    if with_stats:
        return pl.pallas_call(
            _mm_stats_kernel,
            grid=grid,
            in_specs=in_specs,
            out_specs=[
                pl.BlockSpec((RB, Cout), lambda i: (i, 0)),
                pl.BlockSpec((2, Cout), lambda i: (0, 0)),
            ],
            out_shape=[
                jax.ShapeDtypeStruct((N, Cout), jnp.float32),
                jax.ShapeDtypeStruct((2, Cout), jnp.float32),
            ],
            scratch_shapes=[pltpu.VMEM((2, Cout), jnp.float32)],
        )(y, s1, s2, W)
    return pl.pallas_call(
        _mm_kernel,
        grid=grid,
        in_specs=in_specs,
        out_specs=pl.BlockSpec((RB, Cout), lambda i: (i, 0)),
        out_shape=jax.ShapeDtypeStruct((N, Cout), jnp.float32),
    )(y, s1, s2, W)


def _bn_body(h_ref, st_ref, g_ref, b_ref, ninv):
    st = st_ref[...]
    m = st[0:1, :] * ninv
    var = st[1:2, :] * ninv - m * m
    scale = g_ref[...] / jnp.sqrt(var + 1e-5)
    return jax.nn.relu((h_ref[...] - m) * scale + b_ref[...])


def _bn_kernel_full_pool(h_ref, st_ref, g_ref, b_ref, on_ref, op_ref, *, ninv):
    hn = _bn_body(h_ref, st_ref, g_ref, b_ref, ninv)
    on_ref[...] = hn
    R = hn.shape[0]
    op_ref[...] = jnp.mean(hn.reshape(R // 4, 4, -1), axis=1)


def _bn_kernel_pool(h_ref, st_ref, g_ref, b_ref, op_ref, *, ninv):
    hn = _bn_body(h_ref, st_ref, g_ref, b_ref, ninv)
    R = hn.shape[0]
    op_ref[...] = jnp.mean(hn.reshape(R // 4, 4, -1), axis=1)


def _bn_kernel_split(h_ref, st_ref, g_ref, b_ref, on_ref, *, ninv, out_nch):
    hn = _bn_body(h_ref, st_ref, g_ref, b_ref, ninv)
    R, C = hn.shape
    on_ref[...] = jnp.transpose(
        hn.reshape(R, out_nch, C // out_nch), (1, 0, 2))


def _bn_kernel_full(h_ref, st_ref, g_ref, b_ref, on_ref, *, ninv):
    on_ref[...] = _bn_body(h_ref, st_ref, g_ref, b_ref, ninv)


def _tc_bn(h, stats, g, b, V, emit_full, emit_pool, full_nch=None):
    N, Cout = h.shape
    RB = min(V, 1024)
    grid = (N // RB,)
    gb = g.reshape(1, Cout)
    bb = b.reshape(1, Cout)
    ninv = 1.0 / N
    in_specs = [
        pl.BlockSpec((RB, Cout), lambda i: (i, 0)),
        pl.BlockSpec((2, Cout), lambda i: (0, 0)),
        pl.BlockSpec((1, Cout), lambda i: (0, 0)),
        pl.BlockSpec((1, Cout), lambda i: (0, 0)),
    ]
    full_spec = pl.BlockSpec((RB, Cout), lambda i: (i, 0))
    pool_spec = pl.BlockSpec((RB // 4, Cout), lambda i: (i, 0))
    full_shape = jax.ShapeDtypeStruct((N, Cout), jnp.float32)
    pool_shape = jax.ShapeDtypeStruct((N // 4, Cout), jnp.float32)
    if emit_full and emit_pool:
        return pl.pallas_call(
            functools.partial(_bn_kernel_full_pool, ninv=ninv),
            grid=grid, in_specs=in_specs,
            out_specs=[full_spec, pool_spec],
            out_shape=[full_shape, pool_shape],
        )(h, stats, gb, bb)
    if emit_pool:
        return None, pl.pallas_call(
            functools.partial(_bn_kernel_pool, ninv=ninv),
            grid=grid, in_specs=in_specs,
            out_specs=pool_spec, out_shape=pool_shape,
        )(h, stats, gb, bb)
    if full_nch is not None:
        cg = Cout // full_nch
        return pl.pallas_call(
            functools.partial(_bn_kernel_split, ninv=ninv, out_nch=full_nch),
            grid=grid, in_specs=in_specs,
            out_specs=pl.BlockSpec((full_nch, RB, cg), lambda i: (0, i, 0)),
            out_shape=jax.ShapeDtypeStruct((full_nch, N, cg), jnp.float32),
        )(h, stats, gb, bb), None
    return pl.pallas_call(
        functools.partial(_bn_kernel_full, ninv=ninv),
        grid=grid, in_specs=in_specs,
        out_specs=full_spec, out_shape=full_shape,
    )(h, stats, gb, bb), None


def _level_prep(src, dst, w, V):
    E = src.shape[0]
    deg_p = _sc_degree(dst, w, V=V, E=E)
    dis = _tc_dis(deg_p, V)
    return _sc_edge_norm(src, dst, w, dis, V=V, E=E)


def _conv(table3, src, dst, nw, V, C, n_ch=1):
    E = src.shape[0]
    s1 = _sc_matvec(table3, src, dst, nw, V=V, C=C, E=E, n_ch=n_ch)
    s2 = _sc_matvec(s1, src, dst, nw, V=V, C=C, E=E, n_ch=n_ch)
    return s1, s2


def kernel(x, src5, dst5, wt5, src4, dst4, wt4, src3, dst3, wt3, src2, dst2, wt2, src1, dst1, wt1, src0, dst0, wt0, W5a, g5a, b5a, W5b, g5b, b5b, W4, g4, b4, W3, g3, b3, W2, g2, b2, W1, g1, b1, W0):
    B, V5, C0 = x.shape
    vs = [V5 // (4 ** i) for i in range(6)]

    nw5 = _level_prep(src5, dst5, wt5, vs[0])

    t = x.reshape(1, B * vs[0], C0)
    s1, s2 = _conv(t, src5, dst5, nw5, vs[0], C0)
    h, st = _tc_combine(t, s1, s2, W5a, True)
    t5a, _ = _tc_bn(h, st, g5a, b5a, vs[0], True, False, full_nch=2)

    s1, s2 = _conv(t5a, src5, dst5, nw5, vs[0], 32, n_ch=2)
    h, st = _tc_combine(t5a, s1, s2, W5b, True)
    _, t4 = _tc_bn(h, st, g5b, b5b, vs[0], False, True)

    outs = []
    params = [
        (W4, g4, b4, src4, dst4, wt4),
        (W3, g3, b3, src3, dst3, wt3),
        (W2, g2, b2, src2, dst2, wt2),
        (W1, g1, b1, src1, dst1, wt1),
    ]
    t_cur = t4
    for li, (Wl, gl, bl, srcl, dstl, wl) in enumerate(params):
        Vl = vs[1 + li]
        Cin = Wl.shape[0] // 3
        t3 = t_cur.reshape(1, B * Vl, Cin)
        nwl = _level_prep(srcl, dstl, wl, Vl)
        s1, s2 = _conv(t3, srcl, dstl, nwl, Vl, Cin)
        h, st = _tc_combine(t3, s1, s2, Wl, True)
        hfull, t_next = _tc_bn(h, st, gl, bl, Vl, True, True)
        outs.append(hfull.reshape(B, Vl, Wl.shape[1]))
        t_cur = t_next

    V0 = vs[5]
    t3 = t_cur.reshape(1, B * V0, 512)
    nw0 = _level_prep(src0, dst0, wt0, V0)
    s1, s2 = _conv(t3, src0, dst0, nw0, V0, 512)
    h0 = _tc_combine(t3, s1, s2, W0, False)

    h4, h3, h2, h1 = outs
    return (h0.reshape(B, V0, 512), h1, h2, h3, h4)
